# baseline (speedup 1.0000x reference)
import jax
import jax.numpy as jnp
from jax import lax
from jax.experimental import pallas as pl
from jax.experimental.pallas import tpu as pltpu
from jax.experimental.pallas import tpu_sc as plsc

N = 10000
NPAD = 10240
E = 320000
NTILE = 16
ER = 162
EC = 128
ET = ER * EC
EPAD = NTILE * ET
DH = 64
ROWS_T = NPAD // NTILE
NQW = 1280
DEGR = NPAD // DH


def _sc_body(xs, pkr, ewr, y, nrm_hbm, hA, hB, deg_sh,
             dinv, si0, si1, si2, di0, di1, di2, rp0, rp1, rp2,
             rp3, rp4, rp5, rw0, rw1, rw2, rw3, rw4, rw5, zbuf, idb,
             g0, g1, g2, s0, s1, s2, e0, e1, e2, e3, e4, e5):
    c = lax.axis_index("c")
    s = lax.axis_index("s")
    iota16 = lax.iota(jnp.int32, 16)
    zf16 = jnp.zeros((16,), jnp.float32)
    sidxs = (si0, si1, si2)
    didxs = (di0, di1, di2)
    rps = (rp0, rp1, rp2, rp3, rp4, rp5)
    rws = (rw0, rw1, rw2, rw3, rw4, rw5)
    gsems = (g0, g1, g2)
    ssems = (s0, s1, s2)
    esems = (e0, e1, e2, e3, e4, e5)
    my_pk = pkr.at[s]
    my_ew = ewr.at[s]
    my_nrm = nrm_hbm.at[c].at[s]

    for j in range(2):
        for k in range(5):
            idb[j, pl.ds(16 * k, 16)] = iota16 + (80 * j + 16 * k)
    for k in range(5):
        for k2 in range(4):
            zbuf[k, pl.ds(16 * k2, 16)] = zf16

    def estart(r, k, wsrc):
        pltpu.make_async_copy(my_pk.at[r], rps[k], esems[k]).start()
        pltpu.make_async_copy(wsrc.at[r], rws[k], esems[k]).start()

    def ewait(r, k, wsrc):
        pltpu.make_async_copy(my_pk.at[r], rps[k], esems[k]).wait()
        pltpu.make_async_copy(wsrc.at[r], rws[k], esems[k]).wait()

    def phase0(lane_acc, deg2h):
        lane_base = iota16 * NQW
        for half in range(2):
            @pl.loop(0, 4)
            def _qpass(qq):
                lo = 5120 * half + NQW * qq

                @pl.loop(0, 16 * NQW, step=16)
                def _zero(o):
                    lane_acc[pl.ds(o, 16)] = zf16

                estart(0, 0, my_ew)

                @pl.loop(0, ER // 3)
                def _hist3(i):
                    for k in range(3):
                        r = 3 * i + k
                        j = (k + 1) % 3
                        if k == 2:
                            @pl.when(i < ER // 3 - 1)
                            def _pf():
                                estart(r + 1, j, my_ew)
                        else:
                            estart(r + 1, j, my_ew)
                        ewait(r, k, my_ew)
                        ps = [rps[k][pl.ds(16 * q, 16)] for q in range(8)]
                        ws = [rws[k][pl.ds(16 * q, 16)] for q in range(8)]
                        dqs = [(p & 16383) - lo for p in ps]
                        ms = [(dq >= 0) & (dq < NQW) for dq in dqs]
                        for q in range(8):
                            dq = jnp.where(ms[q], dqs[q], 0)
                            plsc.addupdate_scatter(
                                lane_acc, [lane_base + dq], ws[q], mask=ms[q])

                @pl.loop(0, NQW // 16)
                def _red(i):
                    acc = lane_acc[pl.ds(16 * i, 16)]
                    for l in range(1, 16):
                        acc = acc + lane_acc[pl.ds(l * NQW + 16 * i, 16)]
                    deg2h[NQW // DH * qq + (i >> 2),
                          pl.ds((i & 3) * 16, 16)] = acc

            pltpu.sync_copy(zbuf, deg_sh.at[pl.ds(80 * half + 5 * s, 5)])
            plsc.subcore_barrier()
            pltpu.sync_copy(deg2h, deg_sh.at[idb.at[half]], add=True)
            plsc.subcore_barrier()
            pltpu.sync_copy(deg_sh.at[pl.ds(80 * half, 80)], deg2h)

            @pl.loop(0, 80)
            def _rsqrt(i):
                for k in range(4):
                    d = deg2h[i, pl.ds(16 * k, 16)] + 1.0
                    yv = plsc.bitcast(
                        jnp.int32(0x5F3759DF)
                        - (plsc.bitcast(d, jnp.int32) >> 1), jnp.float32)
                    for _ in range(3):
                        yv = yv * (1.5 - 0.5 * d * yv * yv)
                    dinv[pl.ds(5120 * half + DH * i + 16 * k, 16)] = yv

        estart(0, 0, my_ew)

        @pl.loop(0, ER // 3)
        def _norm3(i):
            for k in range(3):
                r = 3 * i + k
                j = (k + 1) % 3
                if k == 2:
                    @pl.when(i < ER // 3 - 1)
                    def _pf():
                        estart(r + 1, j, my_ew)
                else:
                    estart(r + 1, j, my_ew)
                ewait(r, k, my_ew)
                for q in range(8):
                    p = rps[k][pl.ds(16 * q, 16)]
                    w = rws[k][pl.ds(16 * q, 16)]
                    a = plsc.load_gather(dinv, [p >> 14])
                    b = plsc.load_gather(dinv, [p & 16383])
                    rws[k][pl.ds(16 * q, 16)] = a * w * b
                pltpu.sync_copy(rws[k], my_nrm.at[r])

    pl.run_scoped(phase0,
                  pltpu.VMEM((16 * NQW,), jnp.float32),
                  pltpu.VMEM((80, DH), jnp.float32))

    base = s * ROWS_T

    def phase2(buf0, buf1, buf2):
        bufs = (buf0, buf1, buf2)

        def unpack(ke, kb):
            for q in range(8):
                p = rps[ke][pl.ds(16 * q, 16)]
                sidxs[kb][pl.ds(16 * q, 16)] = p >> 14
                didxs[kb][pl.ds(16 * q, 16)] = p & 16383

        def scale(ke, kb):
            @pl.loop(0, EC // 16)
            def _scale(eb):
                nv16 = rws[ke][pl.ds(16 * eb, 16)]
                for l in range(16):
                    e = 16 * eb + l
                    nv = nv16[l]
                    vals = [bufs[kb][e, pl.ds(16 * j, 16)] for j in range(4)]
                    vals = [v * nv for v in vals]
                    for j in range(4):
                        bufs[kb][e, pl.ds(16 * j, 16)] = vals[j]

        def do_round(h_in, h_out):
            for b in range(ROWS_T // EC):
                rb = base + EC * b
                pltpu.sync_copy(h_in.at[pl.ds(rb, EC)], buf0)

                @pl.loop(0, EC // 16)
                def _self(eb):
                    dv16 = dinv[pl.ds(rb + 16 * eb, 16)]
                    sw16 = dv16 * dv16
                    for l in range(16):
                        e = 16 * eb + l
                        sw = sw16[l]
                        vals = [buf0[e, pl.ds(16 * j, 16)] for j in range(4)]
                        vals = [v * sw for v in vals]
                        for j in range(4):
                            buf0[e, pl.ds(16 * j, 16)] = vals[j]

                pltpu.sync_copy(buf0, h_out.at[pl.ds(rb, EC)])
            plsc.subcore_barrier()

            for m in range(5):
                estart(m, m, my_nrm)
            ewait(0, 0, my_nrm)
            unpack(0, 0)
            pltpu.make_async_copy(h_in.at[si0], buf0, g0).start()

            @pl.loop(0, ER // 6)
            def _pipe(i):
                for k in range(6):
                    r = 6 * i + k
                    kb = k % 3
                    jb = (k + 1) % 3
                    ke = k
                    je = (k + 1) % 6
                    pe = (k + 5) % 6
                    if k < 2:
                        @pl.when(i > 0)
                        def _drain():
                            pltpu.make_async_copy(
                                bufs[jb], h_out.at[didxs[jb]],
                                ssems[jb]).wait()
                    else:
                        pltpu.make_async_copy(
                            bufs[jb], h_out.at[didxs[jb]], ssems[jb]).wait()
                    if k == 5:
                        @pl.when(i < ER // 6 - 1)
                        def _g1():
                            ewait(r + 1, je, my_nrm)
                            unpack(je, jb)
                            pltpu.make_async_copy(
                                h_in.at[sidxs[jb]], bufs[jb],
                                gsems[jb]).start()
                    else:
                        ewait(r + 1, je, my_nrm)
                        unpack(je, jb)
                        pltpu.make_async_copy(
                            h_in.at[sidxs[jb]], bufs[jb], gsems[jb]).start()
                    if k == 0:
                        estart(r + 5, pe, my_nrm)
                    else:
                        @pl.when(i < ER // 6 - 1)
                        def _e5():
                            estart(r + 5, pe, my_nrm)
                    pltpu.make_async_copy(h_in.at[sidxs[kb]], bufs[kb],
                                          gsems[kb]).wait()
                    scale(ke, kb)
                    pltpu.make_async_copy(bufs[kb], h_out.at[didxs[kb]],
                                          ssems[kb]).start(add=True)

            for kb in (1, 2):
                pltpu.make_async_copy(bufs[kb], h_out.at[didxs[kb]],
                                      ssems[kb]).wait()
            plsc.subcore_barrier()

        for b in range(ROWS_T // EC):
            rb = base + EC * b
            pltpu.async_copy(xs.at[c].at[pl.ds(rb, EC)], buf0, g0).wait()
            pltpu.sync_copy(buf0, hA.at[pl.ds(rb, EC)])
        plsc.subcore_barrier()

        do_round(hA, hB)
        do_round(hB, hA)
        do_round(hA, hB)

        for b in range(ROWS_T // EC):
            rb = base + EC * b
            pltpu.sync_copy(hB.at[pl.ds(rb, EC)], buf0)
            pltpu.sync_copy(buf0, y.at[c].at[pl.ds(rb, EC)])

    pl.run_scoped(phase2, *([pltpu.VMEM((EC, DH), jnp.float32)] * 3))


_sc_call = pl.kernel(
    _sc_body,
    out_type=[jax.ShapeDtypeStruct((2, NPAD, DH), jnp.float32),
              jax.ShapeDtypeStruct((2, NTILE, ER, EC), jnp.float32)],
    mesh=plsc.VectorSubcoreMesh(core_axis_name="c", subcore_axis_name="s"),
    scratch_types=[
        pltpu.VMEM_SHARED((NPAD, DH), jnp.float32),
        pltpu.VMEM_SHARED((NPAD, DH), jnp.float32),
        pltpu.VMEM_SHARED((DEGR, DH), jnp.float32),
        pltpu.VMEM((NPAD,), jnp.float32),
        pltpu.VMEM((EC,), jnp.int32),
        pltpu.VMEM((EC,), jnp.int32),
        pltpu.VMEM((EC,), jnp.int32),
        pltpu.VMEM((EC,), jnp.int32),
        pltpu.VMEM((EC,), jnp.int32),
        pltpu.VMEM((EC,), jnp.int32),
        pltpu.VMEM((EC,), jnp.int32),
        pltpu.VMEM((EC,), jnp.int32),
        pltpu.VMEM((EC,), jnp.int32),
        pltpu.VMEM((EC,), jnp.int32),
        pltpu.VMEM((EC,), jnp.int32),
        pltpu.VMEM((EC,), jnp.int32),
        pltpu.VMEM((EC,), jnp.float32),
        pltpu.VMEM((EC,), jnp.float32),
        pltpu.VMEM((EC,), jnp.float32),
        pltpu.VMEM((EC,), jnp.float32),
        pltpu.VMEM((EC,), jnp.float32),
        pltpu.VMEM((EC,), jnp.float32),
        pltpu.VMEM((5, DH), jnp.float32),
        pltpu.VMEM((2, 80), jnp.int32),
        pltpu.SemaphoreType.DMA,
        pltpu.SemaphoreType.DMA,
        pltpu.SemaphoreType.DMA,
        pltpu.SemaphoreType.DMA,
        pltpu.SemaphoreType.DMA,
        pltpu.SemaphoreType.DMA,
        pltpu.SemaphoreType.DMA,
        pltpu.SemaphoreType.DMA,
        pltpu.SemaphoreType.DMA,
        pltpu.SemaphoreType.DMA,
        pltpu.SemaphoreType.DMA,
        pltpu.SemaphoreType.DMA,
    ],
    compiler_params=pltpu.CompilerParams(needs_layout_passes=False,
                                         use_tc_tiling_on_sc=False),
    name="gconv_prop_sc",
)


def _tc_body(y_ref, wi_ref, wc_ref, wo_ref, p_ref, hn_ref, cn_ref):
    yv = y_ref[...]
    P = p_ref[...]
    b_i, b_c, b_o, w_co = P[0], P[1], P[2], P[3]
    g_h, bt_h, g_c, bt_c = P[4], P[5], P[6], P[7]
    gi = jax.nn.sigmoid(
        jnp.dot(yv, wi_ref[...], preferred_element_type=jnp.float32) + b_i)
    gt = jnp.tanh(
        jnp.dot(yv, wc_ref[...], preferred_element_type=jnp.float32) + b_c)
    cn = gi * gt
    go = jax.nn.sigmoid(
        jnp.dot(yv, wo_ref[...], preferred_element_type=jnp.float32)
        + w_co * cn + b_o)
    hn = go * jnp.tanh(cn)

    def ln(v, g, b):
        mu = jnp.mean(v, axis=-1, keepdims=True)
        var = jnp.mean((v - mu) * (v - mu), axis=-1, keepdims=True)
        return (v - mu) * lax.rsqrt(var + 1e-5) * g + b

    hn_ref[...] = ln(hn, g_h, bt_h)
    cn_ref[...] = ln(cn, g_c, bt_c)


_BLK = 1024
_tc_call = pl.pallas_call(
    _tc_body,
    grid=(NPAD // _BLK,),
    in_specs=[
        pl.BlockSpec((_BLK, 128), lambda i: (i, 0)),
        pl.BlockSpec((128, 128), lambda i: (0, 0)),
        pl.BlockSpec((128, 128), lambda i: (0, 0)),
        pl.BlockSpec((128, 128), lambda i: (0, 0)),
        pl.BlockSpec((8, 128), lambda i: (0, 0)),
    ],
    out_specs=[
        pl.BlockSpec((_BLK, 128), lambda i: (i, 0)),
        pl.BlockSpec((_BLK, 128), lambda i: (i, 0)),
    ],
    out_shape=[jax.ShapeDtypeStruct((NPAD, 128), jnp.float32)] * 2,
)


def kernel(X, edge_index, edge_weight, W_xi, W_hi, W_xf, W_hf, W_xc, W_hc,
           W_xo, W_ho, b_i, b_f, b_c, b_o, w_ci, w_cf, w_co, g_h, bt_h,
           g_c, bt_c):
    x = X[0]
    xp = jnp.zeros((NPAD, 128), jnp.float32).at[:N].set(x)
    xs = jnp.stack([xp[:, :DH], xp[:, DH:]])
    pk = (edge_index[0] << 14) | edge_index[1]
    pk = jnp.pad(pk, (0, EPAD - E)).reshape(NTILE, ER, EC)
    ew = jnp.pad(edge_weight, (0, EPAD - E)).reshape(NTILE, ER, EC)
    y2, _ = _sc_call(xs, pk, ew)
    Y = jnp.concatenate([y2[0], y2[1]], axis=1)
    P = jnp.stack([b_i, b_c, b_o, w_co, g_h, bt_h, g_c, bt_c])
    Hn, Cn = _tc_call(Y, W_xi, W_xc, W_xo, P)
    return Hn[None, :N], Cn[None, :N]


# speedup vs baseline: 1.0377x; 1.0377x over previous
import jax
import jax.numpy as jnp
from jax import lax
from jax.experimental import pallas as pl
from jax.experimental.pallas import tpu as pltpu
from jax.experimental.pallas import tpu_sc as plsc

N = 10000
NPAD = 10240
E = 320000
NTILE = 16
ER = 159
EC = 128
ET = ER * EC
EPAD = NTILE * ET
DH = 64
ROWS_T = NPAD // NTILE
NQW = 1280
DEGR = NPAD // DH


def _sc_body(xs, pkr, ewr, y, nrm_hbm, hA, hB, deg_sh,
             dinv, si0, si1, si2, di0, di1, di2, rp0, rp1, rp2,
             rw0, rw1, rw2, zbuf, idb,
             g0, g1, g2, s0, s1, s2, e0, e1, e2):
    c = lax.axis_index("c")
    s = lax.axis_index("s")
    iota16 = lax.iota(jnp.int32, 16)
    zf16 = jnp.zeros((16,), jnp.float32)
    sidxs = (si0, si1, si2)
    didxs = (di0, di1, di2)
    rps = (rp0, rp1, rp2)
    rws = (rw0, rw1, rw2)
    gsems = (g0, g1, g2)
    ssems = (s0, s1, s2)
    esems = (e0, e1, e2)
    my_pk = pkr.at[s]
    my_ew = ewr.at[s]
    my_nrm = nrm_hbm.at[c].at[s]

    for j in range(2):
        for k in range(5):
            idb[j, pl.ds(16 * k, 16)] = iota16 + (80 * j + 16 * k)
    for k in range(5):
        for k2 in range(4):
            zbuf[k, pl.ds(16 * k2, 16)] = zf16

    def estart(r, k, wsrc):
        pltpu.make_async_copy(my_pk.at[r], rps[k], esems[k]).start()
        pltpu.make_async_copy(wsrc.at[r], rws[k], esems[k]).start()

    def ewait(r, k, wsrc):
        pltpu.make_async_copy(my_pk.at[r], rps[k], esems[k]).wait()
        pltpu.make_async_copy(wsrc.at[r], rws[k], esems[k]).wait()

    def phase0(lane_acc, deg2h):
        lane_base = iota16 * NQW
        for half in range(2):
            @pl.loop(0, 4)
            def _qpass(qq):
                lo = 5120 * half + NQW * qq

                @pl.loop(0, 16 * NQW, step=16)
                def _zero(o):
                    lane_acc[pl.ds(o, 16)] = zf16

                estart(0, 0, my_ew)

                @pl.loop(0, ER // 3)
                def _hist3(i):
                    for k in range(3):
                        r = 3 * i + k
                        j = (k + 1) % 3
                        if k == 2:
                            @pl.when(i < ER // 3 - 1)
                            def _pf():
                                estart(r + 1, j, my_ew)
                        else:
                            estart(r + 1, j, my_ew)
                        ewait(r, k, my_ew)
                        ps = [rps[k][pl.ds(16 * q, 16)] for q in range(8)]
                        ws = [rws[k][pl.ds(16 * q, 16)] for q in range(8)]
                        dqs = [(p & 16383) - lo for p in ps]
                        ms = [(dq >= 0) & (dq < NQW) for dq in dqs]
                        for q in range(8):
                            dq = jnp.where(ms[q], dqs[q], 0)
                            plsc.addupdate_scatter(
                                lane_acc, [lane_base + dq], ws[q], mask=ms[q])

                @pl.loop(0, NQW // 16)
                def _red(i):
                    acc = lane_acc[pl.ds(16 * i, 16)]
                    for l in range(1, 16):
                        acc = acc + lane_acc[pl.ds(l * NQW + 16 * i, 16)]
                    deg2h[NQW // DH * qq + (i >> 2),
                          pl.ds((i & 3) * 16, 16)] = acc

            pltpu.sync_copy(zbuf, deg_sh.at[pl.ds(80 * half + 5 * s, 5)])
            plsc.subcore_barrier()
            pltpu.sync_copy(deg2h, deg_sh.at[idb.at[half]], add=True)
            plsc.subcore_barrier()
            pltpu.sync_copy(deg_sh.at[pl.ds(80 * half, 80)], deg2h)

            @pl.loop(0, 80)
            def _rsqrt(i):
                for k in range(4):
                    d = deg2h[i, pl.ds(16 * k, 16)] + 1.0
                    yv = plsc.bitcast(
                        jnp.int32(0x5F3759DF)
                        - (plsc.bitcast(d, jnp.int32) >> 1), jnp.float32)
                    for _ in range(3):
                        yv = yv * (1.5 - 0.5 * d * yv * yv)
                    dinv[pl.ds(5120 * half + DH * i + 16 * k, 16)] = yv

        estart(0, 0, my_ew)

        @pl.loop(0, ER // 3)
        def _norm3(i):
            for k in range(3):
                r = 3 * i + k
                j = (k + 1) % 3
                if k == 2:
                    @pl.when(i < ER // 3 - 1)
                    def _pf():
                        estart(r + 1, j, my_ew)
                else:
                    estart(r + 1, j, my_ew)
                ewait(r, k, my_ew)
                for q in range(8):
                    p = rps[k][pl.ds(16 * q, 16)]
                    w = rws[k][pl.ds(16 * q, 16)]
                    a = plsc.load_gather(dinv, [p >> 14])
                    b = plsc.load_gather(dinv, [p & 16383])
                    rws[k][pl.ds(16 * q, 16)] = a * w * b
                pltpu.sync_copy(rws[k], my_nrm.at[r])

    pl.run_scoped(phase0,
                  pltpu.VMEM((16 * NQW,), jnp.float32),
                  pltpu.VMEM((80, DH), jnp.float32))

    base = s * ROWS_T

    def phase2(buf0, buf1, buf2):
        bufs = (buf0, buf1, buf2)

        def unpack(k):
            for q in range(8):
                p = rps[k][pl.ds(16 * q, 16)]
                sidxs[k][pl.ds(16 * q, 16)] = p >> 14
                didxs[k][pl.ds(16 * q, 16)] = p & 16383

        def scale(k):
            @pl.loop(0, EC // 16)
            def _scale(eb):
                nv16 = rws[k][pl.ds(16 * eb, 16)]
                for l in range(16):
                    e = 16 * eb + l
                    nv = nv16[l]
                    vals = [bufs[k][e, pl.ds(16 * j, 16)] for j in range(4)]
                    vals = [v * nv for v in vals]
                    for j in range(4):
                        bufs[k][e, pl.ds(16 * j, 16)] = vals[j]

        def do_round(h_in, h_out):
            for b in range(ROWS_T // EC):
                rb = base + EC * b
                pltpu.sync_copy(h_in.at[pl.ds(rb, EC)], buf0)

                @pl.loop(0, EC // 16)
                def _self(eb):
                    dv16 = dinv[pl.ds(rb + 16 * eb, 16)]
                    sw16 = dv16 * dv16
                    for l in range(16):
                        e = 16 * eb + l
                        sw = sw16[l]
                        vals = [buf0[e, pl.ds(16 * j, 16)] for j in range(4)]
                        vals = [v * sw for v in vals]
                        for j in range(4):
                            buf0[e, pl.ds(16 * j, 16)] = vals[j]

                pltpu.sync_copy(buf0, h_out.at[pl.ds(rb, EC)])
            plsc.subcore_barrier()

            estart(0, 0, my_nrm)
            ewait(0, 0, my_nrm)
            unpack(0)
            pltpu.make_async_copy(h_in.at[si0], buf0, g0).start()
            estart(1, 1, my_nrm)

            @pl.loop(0, ER // 3)
            def _pipe(i):
                for k in range(3):
                    r = 3 * i + k
                    j = (k + 1) % 3
                    jj = (k + 2) % 3
                    if k == 0 or k == 1:
                        @pl.when(i > 0)
                        def _drain():
                            pltpu.make_async_copy(
                                bufs[j], h_out.at[didxs[j]], ssems[j]).wait()
                    else:
                        pltpu.make_async_copy(
                            bufs[j], h_out.at[didxs[j]], ssems[j]).wait()
                    if k == 2:
                        @pl.when(i < ER // 3 - 1)
                        def _g1():
                            ewait(r + 1, j, my_nrm)
                            unpack(j)
                            pltpu.make_async_copy(
                                h_in.at[sidxs[j]], bufs[j], gsems[j]).start()
                    else:
                        ewait(r + 1, j, my_nrm)
                        unpack(j)
                        pltpu.make_async_copy(
                            h_in.at[sidxs[j]], bufs[j], gsems[j]).start()
                    if k == 2:
                        @pl.when(i < ER // 3 - 1)
                        def _e2():
                            estart(r + 2, jj, my_nrm)
                    else:
                        @pl.when(r + 2 < ER)
                        def _e2b():
                            estart(r + 2, jj, my_nrm)
                    pltpu.make_async_copy(h_in.at[sidxs[k]], bufs[k],
                                          gsems[k]).wait()
                    scale(k)
                    pltpu.make_async_copy(bufs[k], h_out.at[didxs[k]],
                                          ssems[k]).start(add=True)

            for k in (1, 2):
                pltpu.make_async_copy(bufs[k], h_out.at[didxs[k]],
                                      ssems[k]).wait()
            plsc.subcore_barrier()

        for b in range(ROWS_T // EC):
            rb = base + EC * b
            pltpu.async_copy(xs.at[c].at[pl.ds(rb, EC)], buf0, g0).wait()
            pltpu.sync_copy(buf0, hA.at[pl.ds(rb, EC)])
        plsc.subcore_barrier()

        do_round(hA, hB)
        do_round(hB, hA)
        do_round(hA, hB)

        for b in range(ROWS_T // EC):
            rb = base + EC * b
            pltpu.sync_copy(hB.at[pl.ds(rb, EC)], buf0)
            pltpu.sync_copy(buf0, y.at[c].at[pl.ds(rb, EC)])

    pl.run_scoped(phase2, *([pltpu.VMEM((EC, DH), jnp.float32)] * 3))


_sc_call = pl.kernel(
    _sc_body,
    out_type=[jax.ShapeDtypeStruct((2, NPAD, DH), jnp.float32),
              jax.ShapeDtypeStruct((2, NTILE, ER, EC), jnp.float32)],
    mesh=plsc.VectorSubcoreMesh(core_axis_name="c", subcore_axis_name="s"),
    scratch_types=[
        pltpu.VMEM_SHARED((NPAD, DH), jnp.float32),
        pltpu.VMEM_SHARED((NPAD, DH), jnp.float32),
        pltpu.VMEM_SHARED((DEGR, DH), jnp.float32),
        pltpu.VMEM((NPAD,), jnp.float32),
        pltpu.VMEM((EC,), jnp.int32),
        pltpu.VMEM((EC,), jnp.int32),
        pltpu.VMEM((EC,), jnp.int32),
        pltpu.VMEM((EC,), jnp.int32),
        pltpu.VMEM((EC,), jnp.int32),
        pltpu.VMEM((EC,), jnp.int32),
        pltpu.VMEM((EC,), jnp.int32),
        pltpu.VMEM((EC,), jnp.int32),
        pltpu.VMEM((EC,), jnp.int32),
        pltpu.VMEM((EC,), jnp.float32),
        pltpu.VMEM((EC,), jnp.float32),
        pltpu.VMEM((EC,), jnp.float32),
        pltpu.VMEM((5, DH), jnp.float32),
        pltpu.VMEM((2, 80), jnp.int32),
        pltpu.SemaphoreType.DMA,
        pltpu.SemaphoreType.DMA,
        pltpu.SemaphoreType.DMA,
        pltpu.SemaphoreType.DMA,
        pltpu.SemaphoreType.DMA,
        pltpu.SemaphoreType.DMA,
        pltpu.SemaphoreType.DMA,
        pltpu.SemaphoreType.DMA,
        pltpu.SemaphoreType.DMA,
    ],
    compiler_params=pltpu.CompilerParams(needs_layout_passes=False,
                                         use_tc_tiling_on_sc=False),
    name="gconv_prop_sc",
)


def _tc_body(y_ref, wi_ref, wc_ref, wo_ref, p_ref, hn_ref, cn_ref):
    yv = y_ref[...]
    P = p_ref[...]
    b_i, b_c, b_o, w_co = P[0], P[1], P[2], P[3]
    g_h, bt_h, g_c, bt_c = P[4], P[5], P[6], P[7]
    gi = jax.nn.sigmoid(
        jnp.dot(yv, wi_ref[...], preferred_element_type=jnp.float32) + b_i)
    gt = jnp.tanh(
        jnp.dot(yv, wc_ref[...], preferred_element_type=jnp.float32) + b_c)
    cn = gi * gt
    go = jax.nn.sigmoid(
        jnp.dot(yv, wo_ref[...], preferred_element_type=jnp.float32)
        + w_co * cn + b_o)
    hn = go * jnp.tanh(cn)

    def ln(v, g, b):
        mu = jnp.mean(v, axis=-1, keepdims=True)
        var = jnp.mean((v - mu) * (v - mu), axis=-1, keepdims=True)
        return (v - mu) * lax.rsqrt(var + 1e-5) * g + b

    hn_ref[...] = ln(hn, g_h, bt_h)
    cn_ref[...] = ln(cn, g_c, bt_c)


_BLK = 1024
_tc_call = pl.pallas_call(
    _tc_body,
    grid=(NPAD // _BLK,),
    in_specs=[
        pl.BlockSpec((_BLK, 128), lambda i: (i, 0)),
        pl.BlockSpec((128, 128), lambda i: (0, 0)),
        pl.BlockSpec((128, 128), lambda i: (0, 0)),
        pl.BlockSpec((128, 128), lambda i: (0, 0)),
        pl.BlockSpec((8, 128), lambda i: (0, 0)),
    ],
    out_specs=[
        pl.BlockSpec((_BLK, 128), lambda i: (i, 0)),
        pl.BlockSpec((_BLK, 128), lambda i: (i, 0)),
    ],
    out_shape=[jax.ShapeDtypeStruct((NPAD, 128), jnp.float32)] * 2,
)


def kernel(X, edge_index, edge_weight, W_xi, W_hi, W_xf, W_hf, W_xc, W_hc,
           W_xo, W_ho, b_i, b_f, b_c, b_o, w_ci, w_cf, w_co, g_h, bt_h,
           g_c, bt_c):
    x = X[0]
    xp = jnp.zeros((NPAD, 128), jnp.float32).at[:N].set(x)
    xs = jnp.stack([xp[:, :DH], xp[:, DH:]])
    pk = (edge_index[0] << 14) | edge_index[1]
    pk = jnp.pad(pk, (0, EPAD - E)).reshape(NTILE, ER, EC)
    ew = jnp.pad(edge_weight, (0, EPAD - E)).reshape(NTILE, ER, EC)
    y2, _ = _sc_call(xs, pk, ew)
    Y = jnp.concatenate([y2[0], y2[1]], axis=1)
    P = jnp.stack([b_i, b_c, b_o, w_co, g_h, bt_h, g_c, bt_c])
    Hn, Cn = _tc_call(Y, W_xi, W_xc, W_xo, P)
    return Hn[None, :N], Cn[None, :N]
